# baseline (speedup 1.0000x reference)
import jax
import jax.numpy as jnp
from jax.experimental import pallas as pl
from jax.experimental.pallas import tpu as pltpu

_BF = jnp.bfloat16
_F8 = jnp.float8_e4m3fn
_F32 = jnp.float32

_YSCALE = 2.0 ** -11
_VMEM = 63 * 1024 * 1024


def _pick_bm(n: int, cap: int) -> int:
    best = 8
    for bm in range(8, min(n, cap) + 1, 8):
        if n % bm == 0:
            best = bm
    return best


def _supp_body(x_ref, w1_ref, supp_ref):
    xb = x_ref[...].astype(_BF)
    wb = w1_ref[...].astype(_BF)
    s = jnp.dot(xb, wb, preferred_element_type=_F32).astype(_BF)
    nones = supp_ref.shape[1] - w1_ref.shape[1]
    ones = jnp.ones(s.shape[:1] + (nones,), _BF)
    supp_ref[...] = jnp.concatenate([s, ones], axis=1)


def _pass_a_body(adj_ref, supp_ref, b1_ref, w2_ref, w3_ref,
                 g_ref, rs_ref, dg_ref, adjb_ref):
    i = pl.program_id(0)
    h = b1_ref.shape[1]
    a32 = adj_ref[...]
    adjb_ref[...] = a32.astype(_F8)
    acc = jnp.dot(a32.astype(_BF), supp_ref[...], preferred_element_type=_F32)
    h1 = jnp.maximum(acc[:, :h] + b1_ref[...], 0.0)
    rs_ref[...] = acc[:, h:]
    w23 = jnp.dot(w2_ref[...].astype(_BF), w3_ref[...].astype(_BF),
                  preferred_element_type=_F32)
    g = jnp.dot(h1.astype(_BF), w23.astype(_BF), preferred_element_type=_F32)
    g8 = g.astype(_F8)
    g_ref[...] = g8

    @pl.when(i == 0)
    def _():
        dg_ref[...] = jnp.zeros_like(dg_ref)

    dg_ref[...] += jnp.sum(g - g8.astype(_F32), axis=0, keepdims=True)


def _pass_b_body(adjb_ref, g_ref, rs_ref, dg_ref, y_ref, dy_ref):
    i = pl.program_id(0)
    n = adjb_ref.shape[1]
    y = jnp.dot(adjb_ref[...], g_ref[...], preferred_element_type=_F32)
    y = y + rs_ref[...] * (dg_ref[...] * (1.0 / n))
    y8 = (y * _YSCALE).astype(_F8)
    y_ref[...] = y8

    @pl.when(i == 0)
    def _():
        dy_ref[...] = jnp.zeros_like(dy_ref)

    dy_ref[...] += jnp.sum(y - y8.astype(_F32) * (1.0 / _YSCALE),
                           axis=0, keepdims=True)


def _pass_c_body(adjb_ref, y_ref, rs_ref, dy_ref, b2_ref, w3_ref, b3_ref,
                 out_ref):
    n = adjb_ref.shape[1]
    p = jnp.dot(adjb_ref[...], y_ref[...],
                preferred_element_type=_F32) * (1.0 / _YSCALE)
    b23 = jnp.dot(b2_ref[...].astype(_BF), w3_ref[...].astype(_BF),
                  preferred_element_type=_F32)
    t = p + rs_ref[...] * (dy_ref[...] * (1.0 / n) + b23) + b3_ref[...]
    m = jnp.max(t, axis=1, keepdims=True)
    lse = jnp.log(jnp.sum(jnp.exp(t - m), axis=1, keepdims=True)) + m
    out_ref[...] = t - lse


def kernel(x, adj, W1, b1, W2, b2, W3, b3):
    n, f = x.shape
    h = W1.shape[1]
    c = W3.shape[1]
    bm = _pick_bm(n, 400)
    nb = n // bm
    bm2 = _pick_bm(n, 2000)
    nb2 = n // bm2
    b1r = b1.reshape(1, h)
    b2r = b2.reshape(1, h)
    b3r = b3.reshape(1, c)

    supp = pl.pallas_call(
        _supp_body,
        grid=(1,),
        in_specs=[
            pl.BlockSpec((n, f), lambda i: (0, 0)),
            pl.BlockSpec((f, h), lambda i: (0, 0)),
        ],
        out_specs=pl.BlockSpec((n, h + c), lambda i: (0, 0)),
        out_shape=jax.ShapeDtypeStruct((n, h + c), _BF),
    )(x, W1)

    g, rs, dg, adjb = pl.pallas_call(
        _pass_a_body,
        grid=(nb,),
        in_specs=[
            pl.BlockSpec((bm, n), lambda i: (i, 0)),
            pl.BlockSpec((n, h + c), lambda i: (0, 0)),
            pl.BlockSpec((1, h), lambda i: (0, 0)),
            pl.BlockSpec((h, h), lambda i: (0, 0)),
            pl.BlockSpec((h, c), lambda i: (0, 0)),
        ],
        out_specs=[
            pl.BlockSpec((bm, c), lambda i: (i, 0)),
            pl.BlockSpec((bm, c), lambda i: (i, 0)),
            pl.BlockSpec((1, c), lambda i: (0, 0)),
            pl.BlockSpec((bm, n), lambda i: (i, 0)),
        ],
        out_shape=[
            jax.ShapeDtypeStruct((n, c), _F8),
            jax.ShapeDtypeStruct((n, c), _F32),
            jax.ShapeDtypeStruct((1, c), _F32),
            jax.ShapeDtypeStruct((n, n), _F8),
        ],
        compiler_params=pltpu.CompilerParams(
            dimension_semantics=("arbitrary",),
            vmem_limit_bytes=_VMEM,
        ),
    )(adj, supp, b1r, W2, W3)

    y, dy = pl.pallas_call(
        _pass_b_body,
        grid=(nb2,),
        in_specs=[
            pl.BlockSpec((bm2, n), lambda i: (i, 0)),
            pl.BlockSpec((n, c), lambda i: (0, 0)),
            pl.BlockSpec((bm2, c), lambda i: (i, 0)),
            pl.BlockSpec((1, c), lambda i: (0, 0)),
        ],
        out_specs=[
            pl.BlockSpec((bm2, c), lambda i: (i, 0)),
            pl.BlockSpec((1, c), lambda i: (0, 0)),
        ],
        out_shape=[
            jax.ShapeDtypeStruct((n, c), _F8),
            jax.ShapeDtypeStruct((1, c), _F32),
        ],
        compiler_params=pltpu.CompilerParams(
            dimension_semantics=("arbitrary",),
            vmem_limit_bytes=_VMEM,
        ),
    )(adjb, g, rs, dg)

    out = pl.pallas_call(
        _pass_c_body,
        grid=(nb2,),
        in_specs=[
            pl.BlockSpec((bm2, n), lambda i: (i, 0)),
            pl.BlockSpec((n, c), lambda i: (0, 0)),
            pl.BlockSpec((bm2, c), lambda i: (i, 0)),
            pl.BlockSpec((1, c), lambda i: (0, 0)),
            pl.BlockSpec((1, h), lambda i: (0, 0)),
            pl.BlockSpec((h, c), lambda i: (0, 0)),
            pl.BlockSpec((1, c), lambda i: (0, 0)),
        ],
        out_specs=pl.BlockSpec((bm2, c), lambda i: (i, 0)),
        out_shape=jax.ShapeDtypeStruct((n, c), _F32),
        compiler_params=pltpu.CompilerParams(
            dimension_semantics=("arbitrary",),
            vmem_limit_bytes=_VMEM,
        ),
    )(adjb, y, rs, dy, b2r, W3, b3r)

    return out


# speedup vs baseline: 1.0876x; 1.0876x over previous
import jax
import jax.numpy as jnp
from jax.experimental import pallas as pl
from jax.experimental.pallas import tpu as pltpu

_BF = jnp.bfloat16
_F8 = jnp.float8_e4m3fn
_F32 = jnp.float32

_YSCALE = 2.0 ** -11
_VMEM = 58 * 1024 * 1024


def _pick_bm(n: int, cap: int) -> int:
    best = 8
    for bm in range(8, min(n, cap) + 1, 8):
        if n % bm == 0:
            best = bm
    return best


def _pass_a_body(adj_ref, x_ref, w1_ref, b1_ref, w2_ref, w3_ref,
                 g_ref, rs_ref, dg_ref, adjb_ref, supp_ref):
    i = pl.program_id(0)
    h = b1_ref.shape[1]

    @pl.when(i == 0)
    def _():
        xb = x_ref[...].astype(_BF)
        wb = w1_ref[...].astype(_BF)
        s = jnp.dot(xb, wb, preferred_element_type=_F32).astype(_BF)
        nones = supp_ref.shape[1] - w1_ref.shape[1]
        ones = jnp.ones(s.shape[:1] + (nones,), _BF)
        supp_ref[...] = jnp.concatenate([s, ones], axis=1)

    a32 = adj_ref[...]
    adjb_ref[...] = a32.astype(_F8)
    acc = jnp.dot(a32.astype(_BF), supp_ref[...], preferred_element_type=_F32)
    h1 = jnp.maximum(acc[:, :h] + b1_ref[...], 0.0)
    rs_ref[...] = acc[:, h:]
    w23 = jnp.dot(w2_ref[...].astype(_BF), w3_ref[...].astype(_BF),
                  preferred_element_type=_F32)
    g = jnp.dot(h1.astype(_BF), w23.astype(_BF), preferred_element_type=_F32)
    g8 = g.astype(_F8)
    g_ref[...] = g8

    @pl.when(i == 0)
    def _():
        dg_ref[...] = jnp.zeros_like(dg_ref)

    dg_ref[...] += jnp.sum(g - g8.astype(_F32), axis=0, keepdims=True)


def _pass_b_body(adjb_ref, g_ref, rs_ref, dg_ref, y_ref, dy_ref):
    i = pl.program_id(0)
    n = adjb_ref.shape[1]
    y = jnp.dot(adjb_ref[...], g_ref[...], preferred_element_type=_F32)
    y = y + rs_ref[...] * (dg_ref[...] * (1.0 / n))
    y8 = (y * _YSCALE).astype(_F8)
    y_ref[...] = y8

    @pl.when(i == 0)
    def _():
        dy_ref[...] = jnp.zeros_like(dy_ref)

    dy_ref[...] += jnp.sum(y - y8.astype(_F32) * (1.0 / _YSCALE),
                           axis=0, keepdims=True)


def _pass_c_body(adjb_ref, y_ref, rs_ref, dy_ref, b2_ref, w3_ref, b3_ref,
                 out_ref):
    n = adjb_ref.shape[1]
    p = jnp.dot(adjb_ref[...], y_ref[...],
                preferred_element_type=_F32) * (1.0 / _YSCALE)
    b23 = jnp.dot(b2_ref[...].astype(_BF), w3_ref[...].astype(_BF),
                  preferred_element_type=_F32)
    t = p + rs_ref[...] * (dy_ref[...] * (1.0 / n) + b23) + b3_ref[...]
    m = jnp.max(t, axis=1, keepdims=True)
    lse = jnp.log(jnp.sum(jnp.exp(t - m), axis=1, keepdims=True)) + m
    out_ref[...] = t - lse


def kernel(x, adj, W1, b1, W2, b2, W3, b3):
    n, f = x.shape
    h = W1.shape[1]
    c = W3.shape[1]
    bm = _pick_bm(n, 400)
    nb = n // bm
    bm2 = _pick_bm(n, 1000)
    nb2 = n // bm2
    b1r = b1.reshape(1, h)
    b2r = b2.reshape(1, h)
    b3r = b3.reshape(1, c)

    g, rs, dg, adjb = pl.pallas_call(
        _pass_a_body,
        grid=(nb,),
        in_specs=[
            pl.BlockSpec((bm, n), lambda i: (i, 0)),
            pl.BlockSpec((n, f), lambda i: (0, 0)),
            pl.BlockSpec((f, h), lambda i: (0, 0)),
            pl.BlockSpec((1, h), lambda i: (0, 0)),
            pl.BlockSpec((h, h), lambda i: (0, 0)),
            pl.BlockSpec((h, c), lambda i: (0, 0)),
        ],
        scratch_shapes=[pltpu.VMEM((n, h + c), _BF)],
        out_specs=[
            pl.BlockSpec((bm, c), lambda i: (i, 0)),
            pl.BlockSpec((bm, c), lambda i: (i, 0)),
            pl.BlockSpec((1, c), lambda i: (0, 0)),
            pl.BlockSpec((bm, n), lambda i: (i, 0)),
        ],
        out_shape=[
            jax.ShapeDtypeStruct((n, c), _F8),
            jax.ShapeDtypeStruct((n, c), _F32),
            jax.ShapeDtypeStruct((1, c), _F32),
            jax.ShapeDtypeStruct((n, n), _F8),
        ],
        compiler_params=pltpu.CompilerParams(
            dimension_semantics=("arbitrary",),
            vmem_limit_bytes=_VMEM,
        ),
    )(adj, x, W1, b1r, W2, W3)

    y, dy = pl.pallas_call(
        _pass_b_body,
        grid=(nb2,),
        in_specs=[
            pl.BlockSpec((bm2, n), lambda i: (i, 0)),
            pl.BlockSpec((n, c), lambda i: (0, 0)),
            pl.BlockSpec((bm2, c), lambda i: (i, 0)),
            pl.BlockSpec((1, c), lambda i: (0, 0)),
        ],
        out_specs=[
            pl.BlockSpec((bm2, c), lambda i: (i, 0)),
            pl.BlockSpec((1, c), lambda i: (0, 0)),
        ],
        out_shape=[
            jax.ShapeDtypeStruct((n, c), _F8),
            jax.ShapeDtypeStruct((1, c), _F32),
        ],
        compiler_params=pltpu.CompilerParams(
            dimension_semantics=("arbitrary",),
            vmem_limit_bytes=_VMEM,
        ),
    )(adjb, g, rs, dg)

    out = pl.pallas_call(
        _pass_c_body,
        grid=(nb2,),
        in_specs=[
            pl.BlockSpec((bm2, n), lambda i: (i, 0)),
            pl.BlockSpec((n, c), lambda i: (0, 0)),
            pl.BlockSpec((bm2, c), lambda i: (i, 0)),
            pl.BlockSpec((1, c), lambda i: (0, 0)),
            pl.BlockSpec((1, h), lambda i: (0, 0)),
            pl.BlockSpec((h, c), lambda i: (0, 0)),
            pl.BlockSpec((1, c), lambda i: (0, 0)),
        ],
        out_specs=pl.BlockSpec((bm2, c), lambda i: (i, 0)),
        out_shape=jax.ShapeDtypeStruct((n, c), _F32),
        compiler_params=pltpu.CompilerParams(
            dimension_semantics=("arbitrary",),
            vmem_limit_bytes=_VMEM,
        ),
    )(adjb, y, rs, dy, b2r, W3, b3r)

    return out


# speedup vs baseline: 1.1289x; 1.0380x over previous
import jax
import jax.numpy as jnp
from jax.experimental import pallas as pl
from jax.experimental.pallas import tpu as pltpu

_BF = jnp.bfloat16
_F8 = jnp.float8_e4m3fn
_F32 = jnp.float32

_YSCALE = 2.0 ** -11
_VMEM = 58 * 1024 * 1024


def _pick_bm(n: int, cap: int) -> int:
    best = 8
    for bm in range(8, min(n, cap) + 1, 8):
        if n % bm == 0:
            best = bm
    return best


def _pass_a_body(adj_ref, x_ref, w1_ref, b1_ref, w2_ref, w3_ref,
                 g_ref, rs_ref, dg_ref, adjb_ref, supp_ref):
    i = pl.program_id(0)
    h = b1_ref.shape[1]

    @pl.when(i == 0)
    def _():
        xb = x_ref[...].astype(_BF)
        wb = w1_ref[...].astype(_BF)
        s = jnp.dot(xb, wb, preferred_element_type=_F32).astype(_BF)
        nones = supp_ref.shape[1] - w1_ref.shape[1]
        ones = jnp.ones(s.shape[:1] + (nones,), _BF)
        supp_ref[...] = jnp.concatenate([s, ones], axis=1)

    a32 = adj_ref[...]
    adjb_ref[...] = a32.astype(_F8)
    acc = jnp.dot(a32.astype(_BF), supp_ref[...], preferred_element_type=_F32)
    h1 = jnp.maximum(acc[:, :h] + b1_ref[...], 0.0)
    rs_ref[...] = acc[:, h:]
    w23 = jnp.dot(w2_ref[...].astype(_BF), w3_ref[...].astype(_BF),
                  preferred_element_type=_F32)
    g = jnp.dot(h1.astype(_BF), w23.astype(_BF), preferred_element_type=_F32)
    g8 = g.astype(_F8)
    g_ref[...] = g8

    @pl.when(i == 0)
    def _():
        dg_ref[...] = jnp.zeros_like(dg_ref)

    dg_ref[...] += jnp.sum(g - g8.astype(_F32), axis=0, keepdims=True)


def _pass_bc_body(adjb_ref, g_ref, rs_ref, dg_ref, b2_ref, w3_ref, b3_ref,
                  out_ref, y_ref, dy_ref):
    p = pl.program_id(0)
    i = pl.program_id(1)
    n = adjb_ref.shape[1]
    bm = adjb_ref.shape[0]

    @pl.when(p == 0)
    def _():
        y = jnp.dot(adjb_ref[...], g_ref[...], preferred_element_type=_F32)
        y = y + rs_ref[...] * (dg_ref[...] * (1.0 / n))
        y8 = (y * _YSCALE).astype(_F8)
        y_ref[pl.ds(pl.multiple_of(i * bm, bm), bm), :] = y8

        @pl.when(i == 0)
        def _():
            dy_ref[...] = jnp.zeros_like(dy_ref)

        dy_ref[...] += jnp.sum(y - y8.astype(_F32) * (1.0 / _YSCALE),
                               axis=0, keepdims=True)

    @pl.when(p == 1)
    def _():
        q = jnp.dot(adjb_ref[...], y_ref[...],
                    preferred_element_type=_F32) * (1.0 / _YSCALE)
        b23 = jnp.dot(b2_ref[...].astype(_BF), w3_ref[...].astype(_BF),
                      preferred_element_type=_F32)
        t = q + rs_ref[...] * (dy_ref[...] * (1.0 / n) + b23) + b3_ref[...]
        m = jnp.max(t, axis=1, keepdims=True)
        lse = jnp.log(jnp.sum(jnp.exp(t - m), axis=1, keepdims=True)) + m
        out_ref[...] = t - lse


def kernel(x, adj, W1, b1, W2, b2, W3, b3):
    n, f = x.shape
    h = W1.shape[1]
    c = W3.shape[1]
    bm = _pick_bm(n, 400)
    nb = n // bm
    bm2 = _pick_bm(n, 1000)
    nb2 = n // bm2
    b1r = b1.reshape(1, h)
    b2r = b2.reshape(1, h)
    b3r = b3.reshape(1, c)

    g, rs, dg, adjb = pl.pallas_call(
        _pass_a_body,
        grid=(nb,),
        in_specs=[
            pl.BlockSpec((bm, n), lambda i: (i, 0)),
            pl.BlockSpec((n, f), lambda i: (0, 0)),
            pl.BlockSpec((f, h), lambda i: (0, 0)),
            pl.BlockSpec((1, h), lambda i: (0, 0)),
            pl.BlockSpec((h, h), lambda i: (0, 0)),
            pl.BlockSpec((h, c), lambda i: (0, 0)),
        ],
        scratch_shapes=[pltpu.VMEM((n, h + c), _BF)],
        out_specs=[
            pl.BlockSpec((bm, c), lambda i: (i, 0)),
            pl.BlockSpec((bm, c), lambda i: (i, 0)),
            pl.BlockSpec((1, c), lambda i: (0, 0)),
            pl.BlockSpec((bm, n), lambda i: (i, 0)),
        ],
        out_shape=[
            jax.ShapeDtypeStruct((n, c), _F8),
            jax.ShapeDtypeStruct((n, c), _F32),
            jax.ShapeDtypeStruct((1, c), _F32),
            jax.ShapeDtypeStruct((n, n), _F8),
        ],
        compiler_params=pltpu.CompilerParams(
            dimension_semantics=("arbitrary",),
            vmem_limit_bytes=_VMEM,
        ),
    )(adj, x, W1, b1r, W2, W3)

    out = pl.pallas_call(
        _pass_bc_body,
        grid=(2, nb2),
        in_specs=[
            pl.BlockSpec((bm2, n), lambda p, i: (i, 0)),
            pl.BlockSpec((n, c), lambda p, i: (0, 0)),
            pl.BlockSpec((bm2, c), lambda p, i: (i, 0)),
            pl.BlockSpec((1, c), lambda p, i: (0, 0)),
            pl.BlockSpec((1, h), lambda p, i: (0, 0)),
            pl.BlockSpec((h, c), lambda p, i: (0, 0)),
            pl.BlockSpec((1, c), lambda p, i: (0, 0)),
        ],
        out_specs=pl.BlockSpec((bm2, c), lambda p, i: (p * i, 0)),
        out_shape=jax.ShapeDtypeStruct((n, c), _F32),
        scratch_shapes=[
            pltpu.VMEM((n, c), _F8),
            pltpu.VMEM((1, c), _F32),
        ],
        compiler_params=pltpu.CompilerParams(
            dimension_semantics=("arbitrary", "arbitrary"),
            vmem_limit_bytes=_VMEM,
        ),
    )(adjb, g, rs, dg, b2r, W3, b3r)

    return out
